# Initial kernel scaffold; baseline (speedup 1.0000x reference)
import jax
import jax.numpy as jnp
from jax.experimental import pallas as pl


def kernel(input, hidden_h0, hidden_c0, emb, iq_w, iq_b, ik_w, ik_b, iv_w, iv_b, lstm_wih, lstm_whh, lstm_b, mq_w, mq_b, mk_w, mk_b, mv_w, mv_b, mf_w, mf_b, mg_w, mg_b, dec_w, dec_b):
    raise NotImplementedError("write your pallas kernel here")


# speedup vs baseline: 1.8708x; 1.8708x over previous
import functools

import jax
import jax.numpy as jnp
import numpy as np
from jax import lax
from jax.experimental import pallas as pl
from jax.experimental.pallas import tpu as pltpu
from jax.experimental.pallas import tpu_sc as plsc

NTOKEN = 10000
NINP = 600
NHID = 600
NB = 6
BS = NHID // NB
TOPK = 4
ATT_OUT = BS * 4
DK = 64
MH = 4
MDK = 32
SEQ = 70
BATCH = 64

DPAD = 640
NW = 32
NTOT = SEQ * BATCH
BPAD = 4608
BPW = BPAD // NW


def _sc_gather(table_pad, idx_pad):
  mesh = plsc.VectorSubcoreMesh(core_axis_name="c", subcore_axis_name="s",
                                num_cores=2)

  @functools.partial(
      pl.kernel,
      mesh=mesh,
      out_type=jax.ShapeDtypeStruct((BPAD, DPAD), jnp.float32),
      scratch_types=[
          pltpu.VMEM((BPW,), jnp.int32),
          pltpu.VMEM((BPW, DPAD), jnp.float32),
          pltpu.SemaphoreType.DMA,
      ],
  )
  def k(table_hbm, idx_hbm, out_hbm, idx_v, rows_v, sem):
    wid = lax.axis_index("s") * 2 + lax.axis_index("c")
    base = wid * BPW
    pltpu.sync_copy(idx_hbm.at[pl.ds(base, BPW)], idx_v)
    pltpu.async_copy(table_hbm.at[idx_v], rows_v, sem).wait()
    pltpu.sync_copy(rows_v, out_hbm.at[pl.ds(base, BPW)])

  return k(table_pad, idx_pad)


_F32 = jnp.float32


def _head_seg(rows, cols, transpose=False):
  di = lax.broadcasted_iota(jnp.int32, (rows, cols), 1 if transpose else 0)
  hi = lax.broadcasted_iota(jnp.int32, (rows, cols), 0 if transpose else 1)
  return (di // MDK == hi).astype(_F32)


def _rbf(x):
  return x.astype(jnp.bfloat16).astype(_F32)


def _bdot(a, b):
  return jnp.dot(a.astype(jnp.bfloat16), b.astype(jnp.bfloat16),
                 preferred_element_type=_F32)


def _bdot_t(a, b):
  return lax.dot_general(a.astype(jnp.bfloat16), b.astype(jnp.bfloat16),
                         (((1,), (1,)), ((), ())),
                         preferred_element_type=_F32)


def _step_kernel(x_ref, h0_ref, c0_ref, iqw_ref, iqb_ref, ikw_ref, ikbr_ref,
                 ivw_ref, ivb_ref, wih_ref, whh_ref, lb_ref,
                 mqw_ref, mqb_ref, mkw_ref, mkb_ref, mvw_ref, mvb_ref,
                 mfw_ref, mfb_ref, mgw_ref, mgb_ref,
                 out_ref, h_s, c_s):
  t = pl.program_id(0)

  @pl.when(t == 0)
  def _():
    h_s[...] = h0_ref[...]
    c_s[...] = c0_ref[...]

  xt = x_ref[0]
  kx = _bdot(xt, ikw_ref[...]) + ikbr_ref[...]
  vx = _bdot(xt, ivw_ref[...]) + ivb_ref[...]
  k0 = ikbr_ref[...]
  v0 = ivb_ref[...]

  scale = 1.0 / float(np.sqrt(DK))
  h_old, c_old, h_new_l, c_new_l, a_null_l = [], [], [], [], []
  for n in range(NB):
    sl = slice(n * BS, (n + 1) * BS)
    h_n = h_s[:, sl]
    c_n = c_s[:, sl]
    h_old.append(h_n)
    c_old.append(c_n)
    q_n = _bdot(h_n, iqw_ref[...]) + iqb_ref[...]
    qr = _rbf(q_n)
    s_null = jnp.sum(qr * _rbf(k0), axis=1, keepdims=True) * scale
    s_x = jnp.sum(qr * _rbf(kx), axis=1, keepdims=True) * scale
    sm = jnp.maximum(s_null, s_x)
    e0 = jnp.exp(s_null - sm)
    e1 = jnp.exp(s_x - sm)
    den = e0 + e1
    a_null = e0 / den
    a_x = e1 / den
    a_null_l.append(a_null)
    inp_use = _rbf(a_null) * _rbf(v0) + _rbf(a_x) * _rbf(vx)
    gates = (_bdot_t(inp_use, wih_ref[n]) + _bdot_t(h_n, whh_ref[n])
             + lb_ref[n])
    gi = gates[:, 0:BS]
    gf = gates[:, BS:2 * BS]
    gg = gates[:, 2 * BS:3 * BS]
    go = gates[:, 3 * BS:4 * BS]
    c_new = jax.nn.sigmoid(gf) * c_n + jax.nn.sigmoid(gi) * jnp.tanh(gg)
    h_new = jax.nn.sigmoid(go) * jnp.tanh(c_new)
    c_new_l.append(c_new)
    h_new_l.append(h_new)

  na = jnp.concatenate(a_null_l, axis=1)
  masks = []
  ji = lax.broadcasted_iota(jnp.int32, (1, NB), 1)
  for i in range(NB):
    na_i = na[:, i:i + 1]
    gt = (na > na_i).astype(_F32)
    eq = (na == na_i).astype(_F32) * (ji < i).astype(_F32)
    rank = jnp.sum(gt + eq, axis=1, keepdims=True)
    masks.append((rank >= float(NB - TOPK)).astype(_F32))

  q2 = [_bdot(h, mqw_ref[...]) + mqb_ref[...] for h in h_new_l]
  k2 = [_bdot(h, mkw_ref[...]) + mkb_ref[...] for h in h_new_l]
  v2 = [_bdot(h, mvw_ref[...]) + mvb_ref[...] for h in h_new_l]
  mscale = 1.0 / float(np.sqrt(MDK))
  hsegT = _head_seg(MH, MH * MDK, transpose=True)
  for n in range(NB):
    q2r = _rbf(q2[n])
    s_nm = []
    for m in range(NB):
      prod = q2r * _rbf(k2[m])
      cols = [jnp.sum(prod[:, h * MDK:(h + 1) * MDK], axis=1, keepdims=True)
              * mscale for h in range(MH)]
      s_nm.append(jnp.concatenate(cols, axis=1))
    smax = s_nm[0]
    for m in range(1, NB):
      smax = jnp.maximum(smax, s_nm[m])
    es = [jnp.exp(s - smax) for s in s_nm]
    den = es[0]
    for m in range(1, NB):
      den = den + es[m]
    o2 = None
    for m in range(NB):
      w = _bdot(es[m] / den, hsegT)
      term = _rbf(w) * _rbf(v2[m])
      o2 = term if o2 is None else o2 + term
    upd = (jax.nn.sigmoid(_bdot(o2, mgw_ref[...]) + mgb_ref[...])
           * jnp.tanh(_bdot(o2, mfw_ref[...]) + mfb_ref[...]))
    h2 = h_new_l[n] + upd
    m_n = masks[n]
    h_out = m_n * h2 + (1.0 - m_n) * h_old[n]
    c_out = m_n * c_new_l[n] + (1.0 - m_n) * c_old[n]
    sl = slice(n * BS, (n + 1) * BS)
    h_s[:, sl] = h_out
    c_s[:, sl] = c_out
    out_ref[0, :, sl] = h_out


def _recurrence(x_r, h0, c0, iq_w, iq_b, ik_w_p, ik_b, iv_w_p, iv_b,
                lstm_wih, lstm_whh, lstm_b, mq_w, mq_b, mk_w, mk_b,
                mv_w, mv_b, mf_w, mf_b, mg_w, mg_b):
  def full(shape):
    nzero = len(shape)
    return pl.BlockSpec(shape, lambda t, _n=nzero: (0,) * _n)
  in_specs = [
      pl.BlockSpec((1, BATCH, DPAD), lambda t: (t, 0, 0)),
      full((BATCH, NHID)),
      full((BATCH, NHID)),
      full((BS, DK)),
      full((1, DK)),
      full((DPAD, DK)),
      full((1, DK)),
      full((DPAD, ATT_OUT)),
      full((1, ATT_OUT)),
      full((NB, 4 * BS, ATT_OUT)),
      full((NB, 4 * BS, BS)),
      full((NB, 1, 4 * BS)),
      full((BS, MH * MDK)), full((1, MH * MDK)),
      full((BS, MH * MDK)), full((1, MH * MDK)),
      full((BS, MH * MDK)), full((1, MH * MDK)),
      full((MH * MDK, BS)), full((1, BS)),
      full((MH * MDK, BS)), full((1, BS)),
  ]
  return pl.pallas_call(
      _step_kernel,
      grid=(SEQ,),
      in_specs=in_specs,
      out_specs=pl.BlockSpec((1, BATCH, NHID), lambda t: (t, 0, 0)),
      out_shape=jax.ShapeDtypeStruct((SEQ, BATCH, NHID), jnp.float32),
      scratch_shapes=[
          pltpu.VMEM((BATCH, NHID), jnp.float32),
          pltpu.VMEM((BATCH, NHID), jnp.float32),
      ],
  )(x_r, h0, c0, iq_w, iq_b.reshape(1, DK), ik_w_p, ik_b.reshape(1, DK),
    iv_w_p, iv_b.reshape(1, ATT_OUT), lstm_wih,
    lstm_whh, lstm_b.reshape(NB, 1, 4 * BS), mq_w, mq_b.reshape(1, MH * MDK),
    mk_w, mk_b.reshape(1, MH * MDK), mv_w, mv_b.reshape(1, MH * MDK),
    mf_w, mf_b.reshape(1, BS), mg_w, mg_b.reshape(1, BS))


_MT = 448
_NT = 2048


def _decode_kernel(x_ref, w_ref, b_ref, out_ref):
  x = x_ref[...].astype(jnp.bfloat16)
  w = w_ref[...].astype(jnp.bfloat16)
  acc = jnp.dot(x, w, preferred_element_type=jnp.float32)
  out_ref[...] = acc + b_ref[...]


def _decode(outs2d, dec_w, dec_b):
  grid = (pl.cdiv(NTOKEN, _NT), NTOT // _MT)
  return pl.pallas_call(
      _decode_kernel,
      grid=grid,
      in_specs=[
          pl.BlockSpec((_MT, NHID), lambda n, m: (m, 0)),
          pl.BlockSpec((NHID, _NT), lambda n, m: (0, n)),
          pl.BlockSpec((1, _NT), lambda n, m: (0, n)),
      ],
      out_specs=pl.BlockSpec((_MT, _NT), lambda n, m: (m, n)),
      out_shape=jax.ShapeDtypeStruct((NTOT, NTOKEN), jnp.float32),
  )(outs2d, dec_w, dec_b.reshape(1, NTOKEN))


def kernel(input, hidden_h0, hidden_c0, emb, iq_w, iq_b, ik_w, ik_b, iv_w,
           iv_b, lstm_wih, lstm_whh, lstm_b, mq_w, mq_b, mk_w, mk_b, mv_w,
           mv_b, mf_w, mf_b, mg_w, mg_b, dec_w, dec_b):
  table_pad = jnp.pad(emb, ((0, 0), (0, DPAD - NINP)))
  idx = input.reshape(-1).astype(jnp.int32)
  idx_pad = jnp.pad(idx, (0, BPAD - NTOT))
  x_rows = _sc_gather(table_pad, idx_pad)
  x_r = x_rows.reshape(BPAD // BATCH, BATCH, DPAD)

  ik_w_p = jnp.pad(ik_w, ((0, DPAD - NINP), (0, 0)))
  iv_w_p = jnp.pad(iv_w, ((0, DPAD - NINP), (0, 0)))

  outs = _recurrence(x_r, hidden_h0, hidden_c0, iq_w, iq_b, ik_w_p, ik_b,
                     iv_w_p, iv_b, lstm_wih, lstm_whh, lstm_b, mq_w, mq_b,
                     mk_w, mk_b, mv_w, mv_b, mf_w, mf_b, mg_w, mg_b)

  logits = _decode(outs.reshape(NTOT, NHID), dec_w, dec_b)
  return logits.reshape(SEQ, BATCH, NTOKEN)
